# baseline (speedup 1.0000x reference)
import jax
import jax.numpy as jnp
from jax.experimental import pallas as pl
from jax.experimental.pallas import tpu as pltpu

B, S, D, OUT, E = 2, 4096, 768, 768, 8
TILE = 2048


def _moe_ffn_kernel(x_ref, wg_ref, bg_ref, wa_hbm, ba_ref, wb_hbm, bb_ref,
                    wc_hbm, bc_ref, o_ref, wa_buf, wb_buf, wc_buf, sem):
    def start_copies(e, slot):
        pltpu.make_async_copy(wa_hbm.at[e], wa_buf.at[slot],
                              sem.at[0, slot]).start()
        pltpu.make_async_copy(wb_hbm.at[e], wb_buf.at[slot],
                              sem.at[1, slot]).start()
        pltpu.make_async_copy(wc_hbm.at[e], wc_buf.at[slot],
                              sem.at[2, slot]).start()

    def wait_copies(e, slot):
        pltpu.make_async_copy(wa_hbm.at[e], wa_buf.at[slot],
                              sem.at[0, slot]).wait()
        pltpu.make_async_copy(wb_hbm.at[e], wb_buf.at[slot],
                              sem.at[1, slot]).wait()
        pltpu.make_async_copy(wc_hbm.at[e], wc_buf.at[slot],
                              sem.at[2, slot]).wait()

    start_copies(0, 0)

    x = x_ref[...]
    logits = jnp.dot(x, wg_ref[...], preferred_element_type=jnp.float32)
    gates = jax.nn.softmax(logits + bg_ref[...], axis=-1)
    o_ref[...] = jnp.dot(gates, bc_ref[...],
                         preferred_element_type=jnp.float32)
    lane = jax.lax.broadcasted_iota(jnp.int32, gates.shape, 1)

    def body(e, carry):
        slot = jax.lax.rem(e, 2)

        @pl.when(e + 1 < E)
        def _prefetch():
            start_copies(e + 1, 1 - slot)

        wait_copies(e, slot)
        g_e = jnp.sum(jnp.where(lane == e, gates, 0.0), axis=1,
                      keepdims=True)
        a = jnp.dot(x, wa_buf[slot], preferred_element_type=jnp.float32)
        a = a + ba_ref[e]
        b = jnp.dot(x, wb_buf[slot], preferred_element_type=jnp.float32)
        b = b + bb_ref[e]
        h = (a * jax.lax.logistic(a)) * b
        o_ref[...] += jnp.dot(h * g_e, wc_buf[slot],
                              preferred_element_type=jnp.float32)
        return carry

    jax.lax.fori_loop(0, E, body, 0, unroll=2)


@jax.jit
def _moe_ffn(x, Wg, bg, Wa, ba, Wb, bb, Wc, bc):
    n = x.shape[0]
    grid = (n // TILE,)
    return pl.pallas_call(
        _moe_ffn_kernel,
        grid=grid,
        in_specs=[
            pl.BlockSpec((TILE, D), lambda i: (i, 0)),
            pl.BlockSpec((D, E), lambda i: (0, 0)),
            pl.BlockSpec((1, E), lambda i: (0, 0)),
            pl.BlockSpec(memory_space=pl.ANY),
            pl.BlockSpec((E, 1, OUT), lambda i: (0, 0, 0)),
            pl.BlockSpec(memory_space=pl.ANY),
            pl.BlockSpec((E, 1, OUT), lambda i: (0, 0, 0)),
            pl.BlockSpec(memory_space=pl.ANY),
            pl.BlockSpec((E, OUT), lambda i: (0, 0)),
        ],
        out_specs=pl.BlockSpec((TILE, OUT), lambda i: (i, 0)),
        out_shape=jax.ShapeDtypeStruct((n, OUT), jnp.float32),
        scratch_shapes=[
            pltpu.VMEM((2, D, OUT), jnp.float32),
            pltpu.VMEM((2, D, OUT), jnp.float32),
            pltpu.VMEM((2, OUT, OUT), jnp.float32),
            pltpu.SemaphoreType.DMA((3, 2)),
        ],
    )(x, Wg, bg, Wa, ba, Wb, bb, Wc, bc)


def kernel(inputs, Wg, bg, Wa, ba, Wb, bb, Wc, bc):
    b, s, d = inputs.shape
    x = inputs.reshape(b * s, d)
    out = _moe_ffn(x, Wg, bg.reshape(1, E), Wa, ba.reshape(E, 1, OUT), Wb,
                   bb.reshape(E, 1, OUT), Wc, bc)
    return out.reshape(b, s, OUT)


# speedup vs baseline: 1.0696x; 1.0696x over previous
import jax
import jax.numpy as jnp
from jax.experimental import pallas as pl
from jax.experimental.pallas import tpu as pltpu

B, S, D, OUT, E = 2, 4096, 768, 768, 8
TILE = 2048


def _moe_ffn_kernel(x_ref, wg_ref, bg_ref, wa_hbm, ba_ref, wb_hbm, bb_ref,
                    wc_hbm, bc_ref, o_ref, wa_buf, wb_buf, wc_buf, sem):
    def start_copies(e, slot):
        pltpu.make_async_copy(wa_hbm.at[e], wa_buf.at[slot],
                              sem.at[0, slot]).start()
        pltpu.make_async_copy(wb_hbm.at[e], wb_buf.at[slot],
                              sem.at[1, slot]).start()
        pltpu.make_async_copy(wc_hbm.at[e], wc_buf.at[slot],
                              sem.at[2, slot]).start()

    def wait_copies(e, slot):
        pltpu.make_async_copy(wa_hbm.at[e], wa_buf.at[slot],
                              sem.at[0, slot]).wait()
        pltpu.make_async_copy(wb_hbm.at[e], wb_buf.at[slot],
                              sem.at[1, slot]).wait()
        pltpu.make_async_copy(wc_hbm.at[e], wc_buf.at[slot],
                              sem.at[2, slot]).wait()

    start_copies(0, 0)

    x = x_ref[...]
    logits = jnp.dot(x, wg_ref[...], preferred_element_type=jnp.float32)
    gates = jax.nn.softmax(logits + bg_ref[...], axis=-1)
    o_ref[...] = jnp.dot(gates, bc_ref[...],
                         preferred_element_type=jnp.float32)
    lane = jax.lax.broadcasted_iota(jnp.int32, gates.shape, 1)

    def body(e, carry):
        slot = jax.lax.rem(e, 2)

        @pl.when(e + 1 < E)
        def _prefetch():
            start_copies(e + 1, 1 - slot)

        wait_copies(e, slot)
        g_e = jnp.sum(jnp.where(lane == e, gates, 0.0), axis=1,
                      keepdims=True)
        a = jnp.dot(x, wa_buf[slot], preferred_element_type=jnp.float32)
        a = a + ba_ref[e]
        b = jnp.dot(x, wb_buf[slot], preferred_element_type=jnp.float32)
        b = b + bb_ref[e]
        h = (a * jax.lax.logistic(a)) * b
        o_ref[...] += jnp.dot(h * g_e, wc_buf[slot],
                              preferred_element_type=jnp.float32)
        return carry

    jax.lax.fori_loop(0, E, body, 0, unroll=False)


@jax.jit
def _moe_ffn(x, Wg, bg, Wa, ba, Wb, bb, Wc, bc):
    n = x.shape[0]
    grid = (n // TILE,)
    return pl.pallas_call(
        _moe_ffn_kernel,
        grid=grid,
        in_specs=[
            pl.BlockSpec((TILE, D), lambda i: (i, 0)),
            pl.BlockSpec((D, E), lambda i: (0, 0)),
            pl.BlockSpec((1, E), lambda i: (0, 0)),
            pl.BlockSpec(memory_space=pl.ANY),
            pl.BlockSpec((E, 1, OUT), lambda i: (0, 0, 0)),
            pl.BlockSpec(memory_space=pl.ANY),
            pl.BlockSpec((E, 1, OUT), lambda i: (0, 0, 0)),
            pl.BlockSpec(memory_space=pl.ANY),
            pl.BlockSpec((E, OUT), lambda i: (0, 0)),
        ],
        out_specs=pl.BlockSpec((TILE, OUT), lambda i: (i, 0)),
        out_shape=jax.ShapeDtypeStruct((n, OUT), jnp.float32),
        scratch_shapes=[
            pltpu.VMEM((2, D, OUT), jnp.float32),
            pltpu.VMEM((2, D, OUT), jnp.float32),
            pltpu.VMEM((2, OUT, OUT), jnp.float32),
            pltpu.SemaphoreType.DMA((3, 2)),
        ],
    )(x, Wg, bg, Wa, ba, Wb, bb, Wc, bc)


def kernel(inputs, Wg, bg, Wa, ba, Wb, bb, Wc, bc):
    b, s, d = inputs.shape
    x = inputs.reshape(b * s, d)
    out = _moe_ffn(x, Wg, bg.reshape(1, E), Wa, ba.reshape(E, 1, OUT), Wb,
                   bb.reshape(E, 1, OUT), Wc, bc)
    return out.reshape(b, s, OUT)


# speedup vs baseline: 1.1243x; 1.0512x over previous
import jax
import jax.numpy as jnp
from jax.experimental import pallas as pl
from jax.experimental.pallas import tpu as pltpu

B, S, D, OUT, E = 2, 4096, 768, 768, 8
TILE = 2048


def _moe_ffn_kernel(x_ref, wg_ref, bg_ref, wa_hbm, ba_ref, wb_hbm, bb_ref,
                    wc_hbm, bc_ref, o_ref, wa_buf, wb_buf, wc_buf, sem):
    def start_copies(e, slot):
        pltpu.make_async_copy(wa_hbm.at[e], wa_buf.at[slot],
                              sem.at[0, slot]).start()
        pltpu.make_async_copy(wb_hbm.at[e], wb_buf.at[slot],
                              sem.at[1, slot]).start()
        pltpu.make_async_copy(wc_hbm.at[e], wc_buf.at[slot],
                              sem.at[2, slot]).start()

    def wait_copies(e, slot):
        pltpu.make_async_copy(wa_hbm.at[e], wa_buf.at[slot],
                              sem.at[0, slot]).wait()
        pltpu.make_async_copy(wb_hbm.at[e], wb_buf.at[slot],
                              sem.at[1, slot]).wait()
        pltpu.make_async_copy(wc_hbm.at[e], wc_buf.at[slot],
                              sem.at[2, slot]).wait()

    start_copies(0, 0)

    x = x_ref[...]
    logits = jnp.dot(x, wg_ref[...], preferred_element_type=jnp.float32)
    gates = jax.nn.softmax(logits + bg_ref[...], axis=-1)
    o_ref[...] = jnp.dot(gates, bc_ref[...],
                         preferred_element_type=jnp.float32)
    lane = jax.lax.broadcasted_iota(jnp.int32, gates.shape, 1)

    def body(e, carry):
        slot = jax.lax.rem(e, 2)

        @pl.when(e + 1 < E)
        def _prefetch():
            start_copies(e + 1, 1 - slot)

        wait_copies(e, slot)
        g_e = jnp.sum(jnp.where(lane == e, gates, 0.0), axis=1,
                      keepdims=True)
        a = jnp.dot(x, wa_buf[slot], preferred_element_type=jnp.float32)
        a = a + ba_ref[e]
        b = jnp.dot(x, wb_buf[slot], preferred_element_type=jnp.float32)
        b = b + bb_ref[e]
        sig = 0.5 * (jnp.tanh(0.5 * a) + 1.0)
        h = (a * sig) * b
        o_ref[...] += jnp.dot(h * g_e, wc_buf[slot],
                              preferred_element_type=jnp.float32)
        return carry

    jax.lax.fori_loop(0, E, body, 0, unroll=False)


@jax.jit
def _moe_ffn(x, Wg, bg, Wa, ba, Wb, bb, Wc, bc):
    n = x.shape[0]
    grid = (n // TILE,)
    return pl.pallas_call(
        _moe_ffn_kernel,
        grid=grid,
        in_specs=[
            pl.BlockSpec((TILE, D), lambda i: (i, 0)),
            pl.BlockSpec((D, E), lambda i: (0, 0)),
            pl.BlockSpec((1, E), lambda i: (0, 0)),
            pl.BlockSpec(memory_space=pl.ANY),
            pl.BlockSpec((E, 1, OUT), lambda i: (0, 0, 0)),
            pl.BlockSpec(memory_space=pl.ANY),
            pl.BlockSpec((E, 1, OUT), lambda i: (0, 0, 0)),
            pl.BlockSpec(memory_space=pl.ANY),
            pl.BlockSpec((E, OUT), lambda i: (0, 0)),
        ],
        out_specs=pl.BlockSpec((TILE, OUT), lambda i: (i, 0)),
        out_shape=jax.ShapeDtypeStruct((n, OUT), jnp.float32),
        scratch_shapes=[
            pltpu.VMEM((2, D, OUT), jnp.float32),
            pltpu.VMEM((2, D, OUT), jnp.float32),
            pltpu.VMEM((2, OUT, OUT), jnp.float32),
            pltpu.SemaphoreType.DMA((3, 2)),
        ],
    )(x, Wg, bg, Wa, ba, Wb, bb, Wc, bc)


def kernel(inputs, Wg, bg, Wa, ba, Wb, bb, Wc, bc):
    b, s, d = inputs.shape
    x = inputs.reshape(b * s, d)
    out = _moe_ffn(x, Wg, bg.reshape(1, E), Wa, ba.reshape(E, 1, OUT), Wb,
                   bb.reshape(E, 1, OUT), Wc, bc)
    return out.reshape(b, s, OUT)


# speedup vs baseline: 1.1684x; 1.0392x over previous
import jax
import jax.numpy as jnp
from jax.experimental import pallas as pl
from jax.experimental.pallas import tpu as pltpu

B, S, D, OUT, E = 2, 4096, 768, 768, 8
TILE = 2048


def _moe_ffn_kernel(x_ref, wg_ref, bg_ref, wa_hbm, ba_ref, wb_hbm, bb_ref,
                    wc_hbm, bc_ref, o_ref, wa_buf, wb_buf, wc_buf, sem):
    def start_copies(e, slot):
        pltpu.make_async_copy(wa_hbm.at[e], wa_buf.at[slot],
                              sem.at[0, slot]).start()
        pltpu.make_async_copy(wb_hbm.at[e], wb_buf.at[slot],
                              sem.at[1, slot]).start()
        pltpu.make_async_copy(wc_hbm.at[e], wc_buf.at[slot],
                              sem.at[2, slot]).start()

    def wait_copies(e, slot):
        pltpu.make_async_copy(wa_hbm.at[e], wa_buf.at[slot],
                              sem.at[0, slot]).wait()
        pltpu.make_async_copy(wb_hbm.at[e], wb_buf.at[slot],
                              sem.at[1, slot]).wait()
        pltpu.make_async_copy(wc_hbm.at[e], wc_buf.at[slot],
                              sem.at[2, slot]).wait()

    pid = pl.program_id(0)
    shift = jax.lax.rem(6 * pid, 8)

    @pl.when(pid == 0)
    def _first_tile_prologue():
        start_copies(0, 0)
        start_copies(1, 1)

    x = x_ref[...]
    logits = jnp.dot(x, wg_ref[...], preferred_element_type=jnp.float32)
    gates = jax.nn.softmax(logits + bg_ref[...], axis=-1)
    o_ref[...] = jnp.dot(gates, bc_ref[...],
                         preferred_element_type=jnp.float32)
    lane = jax.lax.broadcasted_iota(jnp.int32, gates.shape, 1)

    def body(p, carry):
        slot = jax.lax.rem(p, 2)
        e = jax.lax.rem(p + shift, 8)

        @pl.when((pid == 0) | (p >= 2))
        def _wait():
            wait_copies(e, slot)

        g_e = jnp.sum(jnp.where(lane == e, gates, 0.0), axis=1,
                      keepdims=True)
        a = jnp.dot(x, wa_buf[slot], preferred_element_type=jnp.float32)
        a = a + ba_ref[e]
        b = jnp.dot(x, wb_buf[slot], preferred_element_type=jnp.float32)
        b = b + bb_ref[e]
        sig = 0.5 * (jnp.tanh(0.5 * a) + 1.0)
        h = (a * sig) * b
        o_ref[...] += jnp.dot(h * g_e, wc_buf[slot],
                              preferred_element_type=jnp.float32)

        @pl.when(p + 2 < E)
        def _prefetch():
            start_copies(jax.lax.rem(p + 2 + shift, 8), slot)

        return carry

    jax.lax.fori_loop(0, E, body, 0, unroll=False)


@jax.jit
def _moe_ffn(x, Wg, bg, Wa, ba, Wb, bb, Wc, bc):
    n = x.shape[0]
    grid = (n // TILE,)
    return pl.pallas_call(
        _moe_ffn_kernel,
        grid=grid,
        in_specs=[
            pl.BlockSpec((TILE, D), lambda i: (i, 0)),
            pl.BlockSpec((D, E), lambda i: (0, 0)),
            pl.BlockSpec((1, E), lambda i: (0, 0)),
            pl.BlockSpec(memory_space=pl.ANY),
            pl.BlockSpec((E, 1, OUT), lambda i: (0, 0, 0)),
            pl.BlockSpec(memory_space=pl.ANY),
            pl.BlockSpec((E, 1, OUT), lambda i: (0, 0, 0)),
            pl.BlockSpec(memory_space=pl.ANY),
            pl.BlockSpec((E, OUT), lambda i: (0, 0)),
        ],
        out_specs=pl.BlockSpec((TILE, OUT), lambda i: (i, 0)),
        out_shape=jax.ShapeDtypeStruct((n, OUT), jnp.float32),
        scratch_shapes=[
            pltpu.VMEM((2, D, OUT), jnp.float32),
            pltpu.VMEM((2, D, OUT), jnp.float32),
            pltpu.VMEM((2, OUT, OUT), jnp.float32),
            pltpu.SemaphoreType.DMA((3, 2)),
        ],
    )(x, Wg, bg, Wa, ba, Wb, bb, Wc, bc)


def kernel(inputs, Wg, bg, Wa, ba, Wb, bb, Wc, bc):
    b, s, d = inputs.shape
    x = inputs.reshape(b * s, d)
    out = _moe_ffn(x, Wg, bg.reshape(1, E), Wa, ba.reshape(E, 1, OUT), Wb,
                   bb.reshape(E, 1, OUT), Wc, bc)
    return out.reshape(b, s, OUT)
